# Initial kernel scaffold; baseline (speedup 1.0000x reference)
import jax
import jax.numpy as jnp
from jax.experimental import pallas as pl


def kernel(a, t, v, edge_index, edge_type, batch, params):
    raise NotImplementedError("write your pallas kernel here")


# speedup vs baseline: 7.4280x; 7.4280x over previous
import functools
import jax
import jax.numpy as jnp
from jax import lax
from jax.experimental import pallas as pl
from jax.experimental.pallas import tpu as pltpu
from jax.experimental.pallas import tpu_sc as plsc

N = 10000
E = 160000
R = 5
B = 64
H = 128

NC = 2
NS = 16
LANES = 16

HBINS = 50048
PAD_BIN = 50040
EPT_H = E // NS
EPT = E // (NC * NS)
KB = 50
NBLK = EPT // KB
SLAB = N // NS
EPTP = 5120
HSLAB = HBINS // NS

def _k2_norm_body(dst_h, et_h, src_h, zseg_h, ones_h, norm_h, gi_h,
                  hist_sh, ebuf1, ebuf2, ebuf3, idx2d, didx, ones_v, rows_v,
                  rbuf, gibuf, sem):
    c = lax.axis_index("c")
    s = lax.axis_index("s")
    wid = s * NC + c

    pltpu.sync_copy(zseg_h, hist_sh.at[pl.ds(s * HSLAB, HSLAB)])
    pltpu.sync_copy(ones_h, ones_v)

    base = s * EPT_H
    pltpu.sync_copy(dst_h.at[pl.ds(base, EPT_H)], ebuf1.at[pl.ds(0, EPT_H)])
    pltpu.sync_copy(et_h.at[pl.ds(base, EPT_H)], ebuf2.at[pl.ds(0, EPT_H)])

    def mk_idx(j, _):
        for k in range(8):
            off = j * 128 + k * 16
            d16 = ebuf1[pl.ds(off, 16)]
            e16 = ebuf2[pl.ds(off, 16)]
            lane = jnp.full((16,), off, jnp.int32) + lax.iota(jnp.int32, 16)
            v = d16 * R + e16
            v = jnp.where(lane < EPT_H, jnp.clip(v, 0, HBINS - 1), PAD_BIN)
            idx2d[j, pl.ds(k * 16, 16)] = v
        return _

    lax.fori_loop(0, 79, mk_idx, None)
    plsc.subcore_barrier()

    def hist_add(j, _):
        pltpu.sync_copy(ones_v, hist_sh.at[idx2d.at[j]], add=True)
        return _

    lax.fori_loop(0, 79, hist_add, None)
    plsc.subcore_barrier()

    def recip_chunk(t, _):
        off = s * HSLAB + t * 512
        pltpu.sync_copy(hist_sh.at[pl.ds(off, 512)], rbuf)

        def recip(i, _):
            v = rbuf[i]
            rbuf[i] = 1.0 / jnp.maximum(v, 1.0)
            return _

        lax.fori_loop(0, 512, recip, None)
        pltpu.sync_copy(rbuf, hist_sh.at[pl.ds(off, 512)])
        return _

    lax.fori_loop(0, HSLAB // 512, recip_chunk, None)
    toff = s * HSLAB + (HSLAB // 512) * 512
    pltpu.sync_copy(hist_sh.at[pl.ds(toff, HSLAB % 512)],
                    rbuf.at[pl.ds(0, HSLAB % 512)])

    def recip_t(i, _):
        v = rbuf[i]
        rbuf[i] = 1.0 / jnp.maximum(v, 1.0)
        return _

    lax.fori_loop(0, HSLAB % 512, recip_t, None)
    pltpu.sync_copy(rbuf.at[pl.ds(0, HSLAB % 512)],
                    hist_sh.at[pl.ds(toff, HSLAB % 512)])
    plsc.subcore_barrier()

    b2 = wid * EPT
    pltpu.sync_copy(dst_h.at[pl.ds(b2, EPT)], ebuf1.at[pl.ds(0, EPT)])
    pltpu.sync_copy(et_h.at[pl.ds(b2, EPT)], ebuf2.at[pl.ds(0, EPT)])
    pltpu.sync_copy(src_h.at[pl.ds(b2, EPT)], ebuf3.at[pl.ds(0, EPT)])

    def mk2(i, _):
        off = i * 16
        d16 = ebuf1[pl.ds(off, 16)]
        e16 = ebuf2[pl.ds(off, 16)]
        s16 = ebuf3[pl.ds(off, 16)]
        lane = jnp.full((16,), off, jnp.int32) + lax.iota(jnp.int32, 16)
        ok = lane < EPT
        v = jnp.where(ok, jnp.clip(d16 * R + e16, 0, HBINS - 1), 0)
        didx[pl.ds(off, 16)] = v
        g = jnp.where(ok, e16 * N + s16, 0)
        gibuf[pl.ds(off, 16)] = g
        return _

    lax.fori_loop(0, 320, mk2, None)

    def deg_blk(j, _):
        pltpu.async_copy(hist_sh.at[didx.at[pl.ds(j * 128, 128)]],
                         rows_v, sem).wait()
        pltpu.sync_copy(rows_v, norm_h.at[pl.ds(wid * EPTP + j * 128, 128)])
        return _

    lax.fori_loop(0, 40, deg_blk, None)
    pltpu.sync_copy(gibuf.at[pl.ds(0, EPT)], gi_h.at[pl.ds(b2, EPT)])


def _k3_agg_body(y_h, gi_h, dst2_h, nrm_h, zeros_h, out_h,
                 agg_sh, gi_v, dst_v, nrm_v, rows_v, sem):
    c = lax.axis_index("c")
    s = lax.axis_index("s")
    wid = s * NC + c

    pltpu.sync_copy(zeros_h, agg_sh.at[pl.ds(s * SLAB, SLAB)])
    pltpu.sync_copy(gi_h.at[pl.ds(wid * NBLK, NBLK)], gi_v)
    pltpu.sync_copy(dst2_h.at[pl.ds(wid * NBLK, NBLK)], dst_v)
    plsc.subcore_barrier()

    def blk(j, _):
        cp = pltpu.async_copy(y_h.at[gi_v.at[j]], rows_v, sem)
        pltpu.sync_copy(nrm_h.at[pl.ds(wid * EPTP + j * KB, KB)], nrm_v)
        cp.wait()
        for k in range(KB):
            nv = nrm_v[k]
            for q in range(H // 16):
                rows_v[k, pl.ds(q * 16, 16)] = rows_v[k, pl.ds(q * 16, 16)] * nv
        pltpu.sync_copy(rows_v, agg_sh.at[dst_v.at[j]], add=True)
        return _

    lax.fori_loop(0, NBLK, blk, None)
    plsc.subcore_barrier()
    pltpu.sync_copy(agg_sh.at[pl.ds(s * SLAB, SLAB)],
                    out_h.at[pl.ds(c * N + s * SLAB, SLAB)])


@functools.lru_cache(maxsize=1)
def _sc_kernels():
    mesh = plsc.VectorSubcoreMesh(core_axis_name="c", subcore_axis_name="s")
    sc_params = pltpu.CompilerParams(use_tc_tiling_on_sc=False)
    k2 = functools.partial(
        pl.kernel,
        mesh=mesh,
        compiler_params=sc_params,
        out_type=[
            jax.ShapeDtypeStruct((NC * NS * EPTP, 16), jnp.float32),
            jax.ShapeDtypeStruct((E,), jnp.int32),
        ],
        scratch_types=[
            pltpu.VMEM_SHARED((HBINS, 16), jnp.float32),
            pltpu.VMEM((10112,), jnp.int32),
            pltpu.VMEM((10112,), jnp.int32),
            pltpu.VMEM((5120,), jnp.int32),
            pltpu.VMEM((79, 128), jnp.int32),
            pltpu.VMEM((5120,), jnp.int32),
            pltpu.VMEM((128, 16), jnp.float32),
            pltpu.VMEM((128, 16), jnp.float32),
            pltpu.VMEM((512, 16), jnp.float32),
            pltpu.VMEM((5120,), jnp.int32),
            pltpu.SemaphoreType.DMA,
        ],
    )(_k2_norm_body)
    k3 = functools.partial(
        pl.kernel,
        mesh=mesh,
        compiler_params=sc_params,
        out_type=jax.ShapeDtypeStruct((NC * N, H), jnp.float32),
        scratch_types=[
            pltpu.VMEM_SHARED((N, H), jnp.float32),
            pltpu.VMEM((NBLK, KB), jnp.int32),
            pltpu.VMEM((NBLK, KB), jnp.int32),
            pltpu.VMEM((KB, 16), jnp.float32),
            pltpu.VMEM((KB, H), jnp.float32),
            pltpu.SemaphoreType.DMA,
        ],
    )(_k3_agg_body)
    return k2, k3


def _ln(x, g, b):
    m = jnp.mean(x, axis=-1, keepdims=True)
    va = jnp.mean((x - m) * (x - m), axis=-1, keepdims=True)
    return (x - m) * jax.lax.rsqrt(va + 1e-5) * g + b


def _lstm(x, w, bsum):
    gates = jnp.dot(x, w, preferred_element_type=jnp.float32) + bsum
    hh = gates.shape[-1] // 4
    i = gates[:, 0:hh]
    g = gates[:, 2 * hh:3 * hh]
    o = gates[:, 3 * hh:4 * hh]
    cc = jax.nn.sigmoid(i) * jnp.tanh(g)
    return jax.nn.sigmoid(o) * jnp.tanh(cc)


def _k1(a_ref, t_ref, v_ref,
        w0f, b0f, w0r, b0r, w1f, b1f, w1r, b1r, alng, alnb,
        tw1, tb1, tlng, tlnb, tw2, tb2,
        vw1, vb1, vlng, vlnb, vw2, vb2,
        wrel, wroot, broot, x_ref, rt_ref, y_ref):
    ab = a_ref[...]
    tb = t_ref[...]
    vb = v_ref[...]
    h0 = jnp.concatenate([_lstm(ab, w0f[...], b0f[...]),
                          _lstm(ab, w0r[...], b0r[...])], axis=-1)
    h1 = jnp.concatenate([_lstm(h0, w1f[...], b1f[...]),
                          _lstm(h0, w1r[...], b1r[...])], axis=-1)
    za = _ln(h1, alng[...], alnb[...])

    th = jnp.dot(tb, tw1[...], preferred_element_type=jnp.float32) + tb1[...]
    th = jax.nn.gelu(_ln(th, tlng[...], tlnb[...]))
    zt = jnp.dot(th, tw2[...], preferred_element_type=jnp.float32) + tb2[...]

    vh = jnp.dot(vb, vw1[...], preferred_element_type=jnp.float32) + vb1[...]
    vh = jax.nn.gelu(_ln(vh, vlng[...], vlnb[...]))
    zv = jnp.dot(vh, vw2[...], preferred_element_type=jnp.float32) + vb2[...]

    x = (za + zt + zv) / 3.0
    x_ref[...] = x
    rt_ref[...] = jnp.dot(x, wroot[...], preferred_element_type=jnp.float32) + broot[...]
    for r in range(R):
        y_ref[r] = jnp.dot(x, wrel[r], preferred_element_type=jnp.float32)


def _k4(p_ref, rt_ref, xin_ref, lng, lnb, wrel, wroot, broot,
        h_ref, rt2_ref, y2_ref):
    sacc = p_ref[0] + p_ref[1] + rt_ref[...]
    h = jax.nn.gelu(_ln(sacc, lng[...], lnb[...])) + xin_ref[...]
    h_ref[...] = h
    rt2_ref[...] = jnp.dot(h, wroot[...], preferred_element_type=jnp.float32) + broot[...]
    for r in range(R):
        y2_ref[r] = jnp.dot(h, wrel[r], preferred_element_type=jnp.float32)


def _k6(p_ref, rt_ref, hin_ref, lng, lnb, bb_ref, wpool, bpool,
        out_ref, sums, maxs, cnts):
    i = pl.program_id(0)

    @pl.when(i == 0)
    def _():
        sums[...] = jnp.zeros((B, H), jnp.float32)
        maxs[...] = jnp.full((B, H), -3.0e38, jnp.float32)
        cnts[...] = jnp.zeros((B, H), jnp.float32)

    sacc = p_ref[0] + p_ref[1] + rt_ref[...]
    h2 = jax.nn.gelu(_ln(sacc, lng[...], lnb[...])) + hin_ref[...]

    bb = bb_ref[...]
    bn = bb.shape[0]
    onehot = (bb == lax.broadcasted_iota(jnp.int32, (bn, B), 1)
              ).astype(jnp.float32)
    dn = (((0,), (0,)), ((), ()))
    sums[...] += lax.dot_general(onehot, h2, dn,
                                 preferred_element_type=jnp.float32)
    cnts[...] += lax.dot_general(onehot, jnp.ones((bn, H), jnp.float32), dn,
                                 preferred_element_type=jnp.float32)

    b0 = jnp.min(bb)
    b1 = jnp.max(bb)

    def seg(b, _):
        mask = bb == b
        mval = jnp.max(jnp.where(mask, h2, -3.0e38), axis=0, keepdims=True)
        maxs[pl.ds(b, 1), :] = jnp.maximum(maxs[pl.ds(b, 1), :], mval)
        return _

    lax.fori_loop(b0, b1 + 1, seg, None)

    @pl.when(i == pl.num_programs(0) - 1)
    def _():
        cnt = cnts[...]
        mean = sums[...] / jnp.maximum(cnt, 1.0)
        mx = jnp.where(cnt > 0.0, maxs[...], 0.0)
        pooled = jnp.concatenate([mean, mx], axis=-1)
        out_ref[...] = jnp.dot(pooled, wpool[...],
                               preferred_element_type=jnp.float32) + bpool[...]


BN = 1000
_GRID = N // BN


def _full(shape):
    return pl.BlockSpec(shape, lambda i: tuple(0 for _ in shape))


def _rows(shape):
    def im(i):
        return (i,) + tuple(0 for _ in shape[1:])
    return pl.BlockSpec(shape, im)


def kernel(a, t, v, edge_index, edge_type, batch, params):
    f32 = jnp.float32
    pa, pt, pv, pg = params['audio'], params['text'], params['visual'], params['gnn']

    src = edge_index[0].astype(jnp.int32)
    dst = edge_index[1].astype(jnp.int32)
    et = edge_type.astype(jnp.int32)
    dst2 = dst.reshape(E // KB, KB)
    bb2 = batch.astype(jnp.int32).reshape(N, 1)

    r2 = lambda b: b.reshape(1, -1).astype(f32)

    k1_in = [
        a.astype(f32), t.astype(f32), v.astype(f32),
        pa['l0f_Wih'].T, r2(pa['l0f_bih'] + pa['l0f_bhh']),
        pa['l0r_Wih'].T, r2(pa['l0r_bih'] + pa['l0r_bhh']),
        pa['l1f_Wih'].T, r2(pa['l1f_bih'] + pa['l1f_bhh']),
        pa['l1r_Wih'].T, r2(pa['l1r_bih'] + pa['l1r_bhh']),
        r2(pa['ln_g']), r2(pa['ln_b']),
        pt['W1'], r2(pt['b1']), r2(pt['ln_g']), r2(pt['ln_b']),
        pt['W2'], r2(pt['b2']),
        pv['W1'], r2(pv['b1']), r2(pv['ln_g']), r2(pv['ln_b']),
        pv['W2'], r2(pv['b2']),
        pg['Wrel1'], pg['Wroot1'], r2(pg['b1']),
    ]
    k1_specs = [
        _rows((BN, a.shape[1])), _rows((BN, t.shape[1])), _rows((BN, v.shape[1])),
        _full((a.shape[1], 256)), _full((1, 256)),
        _full((a.shape[1], 256)), _full((1, 256)),
        _full((H, 256)), _full((1, 256)),
        _full((H, 256)), _full((1, 256)),
        _full((1, H)), _full((1, H)),
        _full((t.shape[1], 256)), _full((1, 256)), _full((1, 256)), _full((1, 256)),
        _full((256, H)), _full((1, H)),
        _full((v.shape[1], H)), _full((1, H)), _full((1, H)), _full((1, H)),
        _full((H, H)), _full((1, H)),
        _full((R, H, H)), _full((H, H)), _full((1, H)),
    ]
    x, rt1, y1 = pl.pallas_call(
        _k1,
        grid=(_GRID,),
        in_specs=k1_specs,
        out_specs=[_rows((BN, H)), _rows((BN, H)),
                   pl.BlockSpec((R, BN, H), lambda i: (0, i, 0))],
        out_shape=[jax.ShapeDtypeStruct((N, H), f32),
                   jax.ShapeDtypeStruct((N, H), f32),
                   jax.ShapeDtypeStruct((R, N, H), f32)],
    )(*k1_in)

    _k2_norm, _k3_agg = _sc_kernels()
    zseg = jnp.zeros((HBINS // NS, 16), f32)
    ones16 = jnp.ones((128, 16), f32)
    norm, gi = _k2_norm(dst, et, src, zseg, ones16)

    zslab = jnp.zeros((SLAB, H), f32)

    gi2 = gi.reshape(E // KB, KB)
    p1 = _k3_agg(y1.reshape(R * N, H), gi2, dst2, norm, zslab)

    k4_specs = [
        pl.BlockSpec((NC, BN, H), lambda i: (0, i, 0)),
        _rows((BN, H)), _rows((BN, H)),
        _full((1, H)), _full((1, H)),
        _full((R, H, H)), _full((H, H)), _full((1, H)),
    ]
    h, rt2, y2 = pl.pallas_call(
        _k4,
        grid=(_GRID,),
        in_specs=k4_specs,
        out_specs=[_rows((BN, H)), _rows((BN, H)),
                   pl.BlockSpec((R, BN, H), lambda i: (0, i, 0))],
        out_shape=[jax.ShapeDtypeStruct((N, H), f32),
                   jax.ShapeDtypeStruct((N, H), f32),
                   jax.ShapeDtypeStruct((R, N, H), f32)],
    )(p1.reshape(NC, N, H), rt1, x, r2(pg['ln1_g']), r2(pg['ln1_b']),
      pg['Wrel2'], pg['Wroot2'], r2(pg['b2']))

    p2 = _k3_agg(y2.reshape(R * N, H), gi2, dst2, norm, zslab)

    k6_specs = [
        pl.BlockSpec((NC, BN, H), lambda i: (0, i, 0)),
        _rows((BN, H)), _rows((BN, H)),
        _full((1, H)), _full((1, H)),
        _rows((BN, 1)),
        _full((2 * H, H)), _full((1, H)),
    ]
    out = pl.pallas_call(
        _k6,
        grid=(_GRID,),
        in_specs=k6_specs,
        out_specs=pl.BlockSpec((B, H), lambda i: (0, 0)),
        out_shape=jax.ShapeDtypeStruct((B, H), f32),
        scratch_shapes=[pltpu.VMEM((B, H), f32)] * 3,
    )(p2.reshape(NC, N, H), rt2, h, r2(pg['ln2_g']), r2(pg['ln2_b']),
      bb2, pg['Wpool'], r2(pg['bpool']))

    return out


# speedup vs baseline: 9.5865x; 1.2906x over previous
import functools
import jax
import jax.numpy as jnp
from jax import lax
from jax.experimental import pallas as pl
from jax.experimental.pallas import tpu as pltpu
from jax.experimental.pallas import tpu_sc as plsc

N = 10000
E = 160000
R = 5
B = 64
H = 128

NC = 2
NS = 16
LANES = 16

HBINS = 50048
PAD_BIN = 50040
EPT_H = E // NS
EPT = E // (NC * NS)
KB = 100
NBLK = EPT // KB
SLAB = N // NS
EPTP = 5120
HSLAB = HBINS // NS

def _k2_norm_body(dst_h, et_h, src_h, zseg_h, ones_h, norm_h, gi_h,
                  hist_sh, ebuf1, ebuf2, ebuf3, idx2d, didx, ones_v, rows_v,
                  rbuf, gibuf, sem):
    c = lax.axis_index("c")
    s = lax.axis_index("s")
    wid = s * NC + c

    pltpu.sync_copy(zseg_h, hist_sh.at[pl.ds(s * HSLAB, HSLAB)])
    pltpu.sync_copy(ones_h, ones_v)

    base = s * EPT_H
    pltpu.sync_copy(dst_h.at[pl.ds(base, EPT_H)], ebuf1.at[pl.ds(0, EPT_H)])
    pltpu.sync_copy(et_h.at[pl.ds(base, EPT_H)], ebuf2.at[pl.ds(0, EPT_H)])

    def mk_idx(j, _):
        for k in range(8):
            off = j * 128 + k * 16
            d16 = ebuf1[pl.ds(off, 16)]
            e16 = ebuf2[pl.ds(off, 16)]
            lane = jnp.full((16,), off, jnp.int32) + lax.iota(jnp.int32, 16)
            v = d16 * R + e16
            v = jnp.where(lane < EPT_H, jnp.clip(v, 0, HBINS - 1), PAD_BIN)
            idx2d[j, pl.ds(k * 16, 16)] = v
        return _

    lax.fori_loop(0, 79, mk_idx, None)
    plsc.subcore_barrier()

    def hist_add(j, _):
        pltpu.sync_copy(ones_v, hist_sh.at[idx2d.at[j]], add=True)
        return _

    lax.fori_loop(0, 79, hist_add, None)
    plsc.subcore_barrier()

    def recip_chunk(t, _):
        off = s * HSLAB + t * 512
        pltpu.sync_copy(hist_sh.at[pl.ds(off, 512)], rbuf)

        def recip(i, _):
            v = rbuf[i]
            rbuf[i] = 1.0 / jnp.maximum(v, 1.0)
            return _

        lax.fori_loop(0, 512, recip, None)
        pltpu.sync_copy(rbuf, hist_sh.at[pl.ds(off, 512)])
        return _

    lax.fori_loop(0, HSLAB // 512, recip_chunk, None)
    toff = s * HSLAB + (HSLAB // 512) * 512
    pltpu.sync_copy(hist_sh.at[pl.ds(toff, HSLAB % 512)],
                    rbuf.at[pl.ds(0, HSLAB % 512)])

    def recip_t(i, _):
        v = rbuf[i]
        rbuf[i] = 1.0 / jnp.maximum(v, 1.0)
        return _

    lax.fori_loop(0, HSLAB % 512, recip_t, None)
    pltpu.sync_copy(rbuf.at[pl.ds(0, HSLAB % 512)],
                    hist_sh.at[pl.ds(toff, HSLAB % 512)])
    plsc.subcore_barrier()

    b2 = wid * EPT
    pltpu.sync_copy(dst_h.at[pl.ds(b2, EPT)], ebuf1.at[pl.ds(0, EPT)])
    pltpu.sync_copy(et_h.at[pl.ds(b2, EPT)], ebuf2.at[pl.ds(0, EPT)])
    pltpu.sync_copy(src_h.at[pl.ds(b2, EPT)], ebuf3.at[pl.ds(0, EPT)])

    def mk2(i, _):
        off = i * 16
        d16 = ebuf1[pl.ds(off, 16)]
        e16 = ebuf2[pl.ds(off, 16)]
        s16 = ebuf3[pl.ds(off, 16)]
        lane = jnp.full((16,), off, jnp.int32) + lax.iota(jnp.int32, 16)
        ok = lane < EPT
        v = jnp.where(ok, jnp.clip(d16 * R + e16, 0, HBINS - 1), 0)
        didx[pl.ds(off, 16)] = v
        g = jnp.where(ok, e16 * N + s16, 0)
        gibuf[pl.ds(off, 16)] = g
        return _

    lax.fori_loop(0, 320, mk2, None)

    def deg_blk(j, _):
        pltpu.async_copy(hist_sh.at[didx.at[pl.ds(j * 128, 128)]],
                         rows_v, sem).wait()
        pltpu.sync_copy(rows_v, norm_h.at[pl.ds(wid * EPTP + j * 128, 128)])
        return _

    lax.fori_loop(0, 40, deg_blk, None)
    pltpu.sync_copy(gibuf.at[pl.ds(0, EPT)], gi_h.at[pl.ds(b2, EPT)])


def _k3_agg_body(y_h, gi_h, dst2_h, nrm_h, zeros_h, out_h,
                 agg_sh, gi_v, dst_v, nrm0, nrm1, rows0, rows1,
                 gsem0, gsem1, nsem0, nsem1):
    c = lax.axis_index("c")
    s = lax.axis_index("s")
    wid = s * NC + c

    rows = (rows0, rows1)
    nrms = (nrm0, nrm1)
    gsems = (gsem0, gsem1)
    nsems = (nsem0, nsem1)

    pltpu.sync_copy(zeros_h, agg_sh.at[pl.ds(s * SLAB, SLAB)])
    pltpu.sync_copy(gi_h.at[pl.ds(wid * NBLK, NBLK)], gi_v)
    pltpu.sync_copy(dst2_h.at[pl.ds(wid * NBLK, NBLK)], dst_v)
    plsc.subcore_barrier()

    def start(j, p):
        pltpu.async_copy(y_h.at[gi_v.at[j]], rows[p], gsems[p])
        pltpu.async_copy(nrm_h.at[pl.ds(wid * EPTP + j * KB, KB)],
                         nrms[p], nsems[p])

    start(0, 0)

    def pair(j2, _):
        for p in (0, 1):
            j = 2 * j2 + p
            start(jnp.minimum(j + 1, NBLK - 1), 1 - p)
            pltpu.make_async_copy(y_h.at[gi_v.at[j]], rows[p],
                                  gsems[p]).wait()
            pltpu.make_async_copy(nrm_h.at[pl.ds(wid * EPTP, KB)],
                                  nrms[p], nsems[p]).wait()
            for k in range(KB):
                nv = nrms[p][k]
                for q in range(H // 16):
                    rows[p][k, pl.ds(q * 16, 16)] = (
                        rows[p][k, pl.ds(q * 16, 16)] * nv)
            pltpu.sync_copy(rows[p], agg_sh.at[dst_v.at[j]], add=True)
        return _

    lax.fori_loop(0, NBLK // 2, pair, None)
    pltpu.make_async_copy(y_h.at[gi_v.at[0]], rows[0], gsems[0]).wait()
    pltpu.make_async_copy(nrm_h.at[pl.ds(wid * EPTP, KB)],
                          nrms[0], nsems[0]).wait()
    plsc.subcore_barrier()
    pltpu.sync_copy(agg_sh.at[pl.ds(s * SLAB, SLAB)],
                    out_h.at[pl.ds(c * N + s * SLAB, SLAB)])


@functools.lru_cache(maxsize=1)
def _sc_kernels():
    mesh = plsc.VectorSubcoreMesh(core_axis_name="c", subcore_axis_name="s")
    sc_params = pltpu.CompilerParams(use_tc_tiling_on_sc=False)
    k2 = functools.partial(
        pl.kernel,
        mesh=mesh,
        compiler_params=sc_params,
        out_type=[
            jax.ShapeDtypeStruct((NC * NS * EPTP, 16), jnp.float32),
            jax.ShapeDtypeStruct((E,), jnp.int32),
        ],
        scratch_types=[
            pltpu.VMEM_SHARED((HBINS, 16), jnp.float32),
            pltpu.VMEM((10112,), jnp.int32),
            pltpu.VMEM((10112,), jnp.int32),
            pltpu.VMEM((5120,), jnp.int32),
            pltpu.VMEM((79, 128), jnp.int32),
            pltpu.VMEM((5120,), jnp.int32),
            pltpu.VMEM((128, 16), jnp.float32),
            pltpu.VMEM((128, 16), jnp.float32),
            pltpu.VMEM((512, 16), jnp.float32),
            pltpu.VMEM((5120,), jnp.int32),
            pltpu.SemaphoreType.DMA,
        ],
    )(_k2_norm_body)
    k3 = functools.partial(
        pl.kernel,
        mesh=mesh,
        compiler_params=sc_params,
        out_type=jax.ShapeDtypeStruct((NC * N, H), jnp.float32),
        scratch_types=[
            pltpu.VMEM_SHARED((N, H), jnp.float32),
            pltpu.VMEM((NBLK, KB), jnp.int32),
            pltpu.VMEM((NBLK, KB), jnp.int32),
            pltpu.VMEM((KB, 16), jnp.float32),
            pltpu.VMEM((KB, 16), jnp.float32),
            pltpu.VMEM((KB, H), jnp.float32),
            pltpu.VMEM((KB, H), jnp.float32),
            pltpu.SemaphoreType.DMA,
            pltpu.SemaphoreType.DMA,
            pltpu.SemaphoreType.DMA,
            pltpu.SemaphoreType.DMA,
        ],
    )(_k3_agg_body)
    return k2, k3


def _ln(x, g, b):
    m = jnp.mean(x, axis=-1, keepdims=True)
    va = jnp.mean((x - m) * (x - m), axis=-1, keepdims=True)
    return (x - m) * jax.lax.rsqrt(va + 1e-5) * g + b


def _lstm(x, w, bsum):
    gates = jnp.dot(x, w, preferred_element_type=jnp.float32) + bsum
    hh = gates.shape[-1] // 4
    i = gates[:, 0:hh]
    g = gates[:, 2 * hh:3 * hh]
    o = gates[:, 3 * hh:4 * hh]
    cc = jax.nn.sigmoid(i) * jnp.tanh(g)
    return jax.nn.sigmoid(o) * jnp.tanh(cc)


def _k1(a_ref, t_ref, v_ref,
        w0f, b0f, w0r, b0r, w1f, b1f, w1r, b1r, alng, alnb,
        tw1, tb1, tlng, tlnb, tw2, tb2,
        vw1, vb1, vlng, vlnb, vw2, vb2,
        wrel, wroot, broot, x_ref, rt_ref, y_ref):
    ab = a_ref[...]
    tb = t_ref[...]
    vb = v_ref[...]
    h0 = jnp.concatenate([_lstm(ab, w0f[...], b0f[...]),
                          _lstm(ab, w0r[...], b0r[...])], axis=-1)
    h1 = jnp.concatenate([_lstm(h0, w1f[...], b1f[...]),
                          _lstm(h0, w1r[...], b1r[...])], axis=-1)
    za = _ln(h1, alng[...], alnb[...])

    th = jnp.dot(tb, tw1[...], preferred_element_type=jnp.float32) + tb1[...]
    th = jax.nn.gelu(_ln(th, tlng[...], tlnb[...]))
    zt = jnp.dot(th, tw2[...], preferred_element_type=jnp.float32) + tb2[...]

    vh = jnp.dot(vb, vw1[...], preferred_element_type=jnp.float32) + vb1[...]
    vh = jax.nn.gelu(_ln(vh, vlng[...], vlnb[...]))
    zv = jnp.dot(vh, vw2[...], preferred_element_type=jnp.float32) + vb2[...]

    x = (za + zt + zv) / 3.0
    x_ref[...] = x
    rt_ref[...] = jnp.dot(x, wroot[...], preferred_element_type=jnp.float32) + broot[...]
    for r in range(R):
        y_ref[r] = jnp.dot(x, wrel[r], preferred_element_type=jnp.float32)


def _k4(p_ref, rt_ref, xin_ref, lng, lnb, wrel, wroot, broot,
        h_ref, rt2_ref, y2_ref):
    sacc = p_ref[0] + p_ref[1] + rt_ref[...]
    h = jax.nn.gelu(_ln(sacc, lng[...], lnb[...])) + xin_ref[...]
    h_ref[...] = h
    rt2_ref[...] = jnp.dot(h, wroot[...], preferred_element_type=jnp.float32) + broot[...]
    for r in range(R):
        y2_ref[r] = jnp.dot(h, wrel[r], preferred_element_type=jnp.float32)


def _k6(p_ref, rt_ref, hin_ref, lng, lnb, bb_ref, wpool, bpool,
        out_ref, sums, maxs, cnts):
    i = pl.program_id(0)

    @pl.when(i == 0)
    def _():
        sums[...] = jnp.zeros((B, H), jnp.float32)
        maxs[...] = jnp.full((B, H), -3.0e38, jnp.float32)
        cnts[...] = jnp.zeros((B, H), jnp.float32)

    sacc = p_ref[0] + p_ref[1] + rt_ref[...]
    h2 = jax.nn.gelu(_ln(sacc, lng[...], lnb[...])) + hin_ref[...]

    bb = bb_ref[...]
    bn = bb.shape[0]
    onehot = (bb == lax.broadcasted_iota(jnp.int32, (bn, B), 1)
              ).astype(jnp.float32)
    dn = (((0,), (0,)), ((), ()))
    sums[...] += lax.dot_general(onehot, h2, dn,
                                 preferred_element_type=jnp.float32)
    cnts[...] += lax.dot_general(onehot, jnp.ones((bn, H), jnp.float32), dn,
                                 preferred_element_type=jnp.float32)

    b0 = jnp.min(bb)
    b1 = jnp.max(bb)

    def seg(b, _):
        mask = bb == b
        mval = jnp.max(jnp.where(mask, h2, -3.0e38), axis=0, keepdims=True)
        maxs[pl.ds(b, 1), :] = jnp.maximum(maxs[pl.ds(b, 1), :], mval)
        return _

    lax.fori_loop(b0, b1 + 1, seg, None)

    @pl.when(i == pl.num_programs(0) - 1)
    def _():
        cnt = cnts[...]
        mean = sums[...] / jnp.maximum(cnt, 1.0)
        mx = jnp.where(cnt > 0.0, maxs[...], 0.0)
        pooled = jnp.concatenate([mean, mx], axis=-1)
        out_ref[...] = jnp.dot(pooled, wpool[...],
                               preferred_element_type=jnp.float32) + bpool[...]


BN = 1000
_GRID = N // BN


def _full(shape):
    return pl.BlockSpec(shape, lambda i: tuple(0 for _ in shape))


def _rows(shape):
    def im(i):
        return (i,) + tuple(0 for _ in shape[1:])
    return pl.BlockSpec(shape, im)


def kernel(a, t, v, edge_index, edge_type, batch, params):
    f32 = jnp.float32
    pa, pt, pv, pg = params['audio'], params['text'], params['visual'], params['gnn']

    src = edge_index[0].astype(jnp.int32)
    dst = edge_index[1].astype(jnp.int32)
    et = edge_type.astype(jnp.int32)
    bb2 = batch.astype(jnp.int32).reshape(N, 1)

    r2 = lambda b: b.reshape(1, -1).astype(f32)

    k1_in = [
        a.astype(f32), t.astype(f32), v.astype(f32),
        pa['l0f_Wih'].T, r2(pa['l0f_bih'] + pa['l0f_bhh']),
        pa['l0r_Wih'].T, r2(pa['l0r_bih'] + pa['l0r_bhh']),
        pa['l1f_Wih'].T, r2(pa['l1f_bih'] + pa['l1f_bhh']),
        pa['l1r_Wih'].T, r2(pa['l1r_bih'] + pa['l1r_bhh']),
        r2(pa['ln_g']), r2(pa['ln_b']),
        pt['W1'], r2(pt['b1']), r2(pt['ln_g']), r2(pt['ln_b']),
        pt['W2'], r2(pt['b2']),
        pv['W1'], r2(pv['b1']), r2(pv['ln_g']), r2(pv['ln_b']),
        pv['W2'], r2(pv['b2']),
        pg['Wrel1'], pg['Wroot1'], r2(pg['b1']),
    ]
    k1_specs = [
        _rows((BN, a.shape[1])), _rows((BN, t.shape[1])), _rows((BN, v.shape[1])),
        _full((a.shape[1], 256)), _full((1, 256)),
        _full((a.shape[1], 256)), _full((1, 256)),
        _full((H, 256)), _full((1, 256)),
        _full((H, 256)), _full((1, 256)),
        _full((1, H)), _full((1, H)),
        _full((t.shape[1], 256)), _full((1, 256)), _full((1, 256)), _full((1, 256)),
        _full((256, H)), _full((1, H)),
        _full((v.shape[1], H)), _full((1, H)), _full((1, H)), _full((1, H)),
        _full((H, H)), _full((1, H)),
        _full((R, H, H)), _full((H, H)), _full((1, H)),
    ]
    x, rt1, y1 = pl.pallas_call(
        _k1,
        grid=(_GRID,),
        in_specs=k1_specs,
        out_specs=[_rows((BN, H)), _rows((BN, H)),
                   pl.BlockSpec((R, BN, H), lambda i: (0, i, 0))],
        out_shape=[jax.ShapeDtypeStruct((N, H), f32),
                   jax.ShapeDtypeStruct((N, H), f32),
                   jax.ShapeDtypeStruct((R, N, H), f32)],
    )(*k1_in)

    _k2_norm, _k3_agg = _sc_kernels()
    zseg = jnp.zeros((HBINS // NS, 16), f32)
    ones16 = jnp.ones((128, 16), f32)
    norm, gi = _k2_norm(dst, et, src, zseg, ones16)

    zslab = jnp.zeros((SLAB, H), f32)

    gi2 = gi.reshape(E // KB, KB)
    dst2 = dst.reshape(E // KB, KB)
    p1 = _k3_agg(y1.reshape(R * N, H), gi2, dst2, norm, zslab)

    k4_specs = [
        pl.BlockSpec((NC, BN, H), lambda i: (0, i, 0)),
        _rows((BN, H)), _rows((BN, H)),
        _full((1, H)), _full((1, H)),
        _full((R, H, H)), _full((H, H)), _full((1, H)),
    ]
    h, rt2, y2 = pl.pallas_call(
        _k4,
        grid=(_GRID,),
        in_specs=k4_specs,
        out_specs=[_rows((BN, H)), _rows((BN, H)),
                   pl.BlockSpec((R, BN, H), lambda i: (0, i, 0))],
        out_shape=[jax.ShapeDtypeStruct((N, H), f32),
                   jax.ShapeDtypeStruct((N, H), f32),
                   jax.ShapeDtypeStruct((R, N, H), f32)],
    )(p1.reshape(NC, N, H), rt1, x, r2(pg['ln1_g']), r2(pg['ln1_b']),
      pg['Wrel2'], pg['Wroot2'], r2(pg['b2']))

    p2 = _k3_agg(y2.reshape(R * N, H), gi2, dst2, norm, zslab)

    k6_specs = [
        pl.BlockSpec((NC, BN, H), lambda i: (0, i, 0)),
        _rows((BN, H)), _rows((BN, H)),
        _full((1, H)), _full((1, H)),
        _rows((BN, 1)),
        _full((2 * H, H)), _full((1, H)),
    ]
    out = pl.pallas_call(
        _k6,
        grid=(_GRID,),
        in_specs=k6_specs,
        out_specs=pl.BlockSpec((B, H), lambda i: (0, 0)),
        out_shape=jax.ShapeDtypeStruct((B, H), f32),
        scratch_shapes=[pltpu.VMEM((B, H), f32)] * 3,
    )(p2.reshape(NC, N, H), rt2, h, r2(pg['ln2_g']), r2(pg['ln2_b']),
      bb2, pg['Wpool'], r2(pg['bpool']))

    return out


# speedup vs baseline: 9.7929x; 1.0215x over previous
import functools
import jax
import jax.numpy as jnp
from jax import lax
from jax.experimental import pallas as pl
from jax.experimental.pallas import tpu as pltpu
from jax.experimental.pallas import tpu_sc as plsc

N = 10000
E = 160000
R = 5
B = 64
H = 128

NC = 2
NS = 16
LANES = 16

HBINS = 50048
PAD_BIN = 50040
EPT_H = E // NS
EPT = E // (NC * NS)
KB = 100
NBLK = EPT // KB
SLAB = N // NS
EPTP = 5120
HSLAB = HBINS // NS

def _k2_norm_body(dst_h, et_h, src_h, zseg_h, ones_h, norm_h, gi_h,
                  hist_sh, ebuf1, ebuf2, ebuf3, idx2d, didx, ones_v, rows_v,
                  rbuf, gibuf, sem, sem2):
    c = lax.axis_index("c")
    s = lax.axis_index("s")
    wid = s * NC + c

    pltpu.sync_copy(zseg_h, hist_sh.at[pl.ds(s * HSLAB, HSLAB)])
    pltpu.sync_copy(ones_h, ones_v)

    base = s * EPT_H
    pltpu.sync_copy(dst_h.at[pl.ds(base, EPT_H)], ebuf1.at[pl.ds(0, EPT_H)])
    pltpu.sync_copy(et_h.at[pl.ds(base, EPT_H)], ebuf2.at[pl.ds(0, EPT_H)])

    def mk_idx(j, _):
        for k in range(8):
            off = j * 128 + k * 16
            d16 = ebuf1[pl.ds(off, 16)]
            e16 = ebuf2[pl.ds(off, 16)]
            lane = jnp.full((16,), off, jnp.int32) + lax.iota(jnp.int32, 16)
            v = d16 * R + e16
            v = jnp.where(lane < EPT_H, jnp.clip(v, 0, HBINS - 1), PAD_BIN)
            idx2d[j, pl.ds(k * 16, 16)] = v
        return _

    lax.fori_loop(0, 79, mk_idx, None)
    plsc.subcore_barrier()

    def hist_add(j, _):
        pltpu.async_copy(ones_v, hist_sh.at[idx2d.at[j]], sem, add=True)
        return _

    lax.fori_loop(0, 79, hist_add, None)

    def hist_drain(j, _):
        pltpu.make_async_copy(ones_v, hist_sh.at[idx2d.at[0]], sem).wait()
        return _

    lax.fori_loop(0, 79, hist_drain, None)
    plsc.subcore_barrier()

    def recip_chunk(t, _):
        off = s * HSLAB + t * 512
        pltpu.sync_copy(hist_sh.at[pl.ds(off, 512)], rbuf)

        def recip(i, _):
            v = rbuf[i]
            rbuf[i] = 1.0 / jnp.maximum(v, 1.0)
            return _

        lax.fori_loop(0, 512, recip, None)
        pltpu.sync_copy(rbuf, hist_sh.at[pl.ds(off, 512)])
        return _

    lax.fori_loop(0, HSLAB // 512, recip_chunk, None)
    toff = s * HSLAB + (HSLAB // 512) * 512
    pltpu.sync_copy(hist_sh.at[pl.ds(toff, HSLAB % 512)],
                    rbuf.at[pl.ds(0, HSLAB % 512)])

    def recip_t(i, _):
        v = rbuf[i]
        rbuf[i] = 1.0 / jnp.maximum(v, 1.0)
        return _

    lax.fori_loop(0, HSLAB % 512, recip_t, None)
    pltpu.sync_copy(rbuf.at[pl.ds(0, HSLAB % 512)],
                    hist_sh.at[pl.ds(toff, HSLAB % 512)])
    plsc.subcore_barrier()

    b2 = wid * EPT
    pltpu.sync_copy(dst_h.at[pl.ds(b2, EPT)], ebuf1.at[pl.ds(0, EPT)])
    pltpu.sync_copy(et_h.at[pl.ds(b2, EPT)], ebuf2.at[pl.ds(0, EPT)])
    pltpu.sync_copy(src_h.at[pl.ds(b2, EPT)], ebuf3.at[pl.ds(0, EPT)])

    def mk2(i, _):
        off = i * 16
        d16 = ebuf1[pl.ds(off, 16)]
        e16 = ebuf2[pl.ds(off, 16)]
        s16 = ebuf3[pl.ds(off, 16)]
        lane = jnp.full((16,), off, jnp.int32) + lax.iota(jnp.int32, 16)
        ok = lane < EPT
        v = jnp.where(ok, jnp.clip(d16 * R + e16, 0, HBINS - 1), 0)
        didx[pl.ds(off, 16)] = v
        g = jnp.where(ok, e16 * N + s16, 0)
        gibuf[pl.ds(off, 16)] = g
        return _

    lax.fori_loop(0, 320, mk2, None)

    nbufs = (rows_v, rbuf)
    dsems = (sem, sem2)

    def startg(j, p):
        pltpu.async_copy(hist_sh.at[didx.at[pl.ds(j * 128, 128)]],
                         nbufs[p].at[pl.ds(0, 128)], dsems[p])

    startg(0, 0)

    def deg_pair(j2, _):
        for p in (0, 1):
            j = 2 * j2 + p
            startg(jnp.minimum(j + 1, 39), 1 - p)
            pltpu.make_async_copy(hist_sh.at[didx.at[pl.ds(0, 128)]],
                                  nbufs[p].at[pl.ds(0, 128)], dsems[p]).wait()
            pltpu.sync_copy(nbufs[p].at[pl.ds(0, 128)],
                            norm_h.at[pl.ds(wid * EPTP + j * 128, 128)])
        return _

    lax.fori_loop(0, 20, deg_pair, None)
    pltpu.make_async_copy(hist_sh.at[didx.at[pl.ds(0, 128)]],
                          nbufs[0].at[pl.ds(0, 128)], dsems[0]).wait()
    pltpu.sync_copy(gibuf.at[pl.ds(0, EPT)], gi_h.at[pl.ds(b2, EPT)])


def _k3_agg_body(y_h, gi_h, dst2_h, nrm_h, zeros_h, out_h,
                 agg_sh, gi_v, dst_v, nrm0, nrm1, rows0, rows1,
                 gsem0, gsem1, nsem0, nsem1):
    c = lax.axis_index("c")
    s = lax.axis_index("s")
    wid = s * NC + c

    rows = (rows0, rows1)
    nrms = (nrm0, nrm1)
    gsems = (gsem0, gsem1)
    nsems = (nsem0, nsem1)

    pltpu.sync_copy(zeros_h, agg_sh.at[pl.ds(s * SLAB, SLAB)])
    pltpu.sync_copy(gi_h.at[pl.ds(wid * NBLK, NBLK)], gi_v)
    pltpu.sync_copy(dst2_h.at[pl.ds(wid * NBLK, NBLK)], dst_v)
    plsc.subcore_barrier()

    def start(j, p):
        pltpu.async_copy(y_h.at[gi_v.at[j]], rows[p], gsems[p])
        pltpu.async_copy(nrm_h.at[pl.ds(wid * EPTP + j * KB, KB)],
                         nrms[p], nsems[p])

    start(0, 0)

    def pair(j2, _):
        for p in (0, 1):
            j = 2 * j2 + p
            start(jnp.minimum(j + 1, NBLK - 1), 1 - p)
            pltpu.make_async_copy(y_h.at[gi_v.at[j]], rows[p],
                                  gsems[p]).wait()
            pltpu.make_async_copy(nrm_h.at[pl.ds(wid * EPTP, KB)],
                                  nrms[p], nsems[p]).wait()
            for k in range(KB):
                nv = nrms[p][k]
                for q in range(H // 16):
                    rows[p][k, pl.ds(q * 16, 16)] = (
                        rows[p][k, pl.ds(q * 16, 16)] * nv)
            pltpu.sync_copy(rows[p], agg_sh.at[dst_v.at[j]], add=True)
        return _

    lax.fori_loop(0, NBLK // 2, pair, None)
    pltpu.make_async_copy(y_h.at[gi_v.at[0]], rows[0], gsems[0]).wait()
    pltpu.make_async_copy(nrm_h.at[pl.ds(wid * EPTP, KB)],
                          nrms[0], nsems[0]).wait()
    plsc.subcore_barrier()
    pltpu.sync_copy(agg_sh.at[pl.ds(s * SLAB, SLAB)],
                    out_h.at[pl.ds(c * N + s * SLAB, SLAB)])


@functools.lru_cache(maxsize=1)
def _sc_kernels():
    mesh = plsc.VectorSubcoreMesh(core_axis_name="c", subcore_axis_name="s")
    sc_params = pltpu.CompilerParams(use_tc_tiling_on_sc=False)
    k2 = functools.partial(
        pl.kernel,
        mesh=mesh,
        compiler_params=sc_params,
        out_type=[
            jax.ShapeDtypeStruct((NC * NS * EPTP, 16), jnp.float32),
            jax.ShapeDtypeStruct((E,), jnp.int32),
        ],
        scratch_types=[
            pltpu.VMEM_SHARED((HBINS, 16), jnp.float32),
            pltpu.VMEM((10112,), jnp.int32),
            pltpu.VMEM((10112,), jnp.int32),
            pltpu.VMEM((5120,), jnp.int32),
            pltpu.VMEM((79, 128), jnp.int32),
            pltpu.VMEM((5120,), jnp.int32),
            pltpu.VMEM((128, 16), jnp.float32),
            pltpu.VMEM((128, 16), jnp.float32),
            pltpu.VMEM((512, 16), jnp.float32),
            pltpu.VMEM((5120,), jnp.int32),
            pltpu.SemaphoreType.DMA,
            pltpu.SemaphoreType.DMA,
        ],
    )(_k2_norm_body)
    k3 = functools.partial(
        pl.kernel,
        mesh=mesh,
        compiler_params=sc_params,
        out_type=jax.ShapeDtypeStruct((NC * N, H), jnp.float32),
        scratch_types=[
            pltpu.VMEM_SHARED((N, H), jnp.float32),
            pltpu.VMEM((NBLK, KB), jnp.int32),
            pltpu.VMEM((NBLK, KB), jnp.int32),
            pltpu.VMEM((KB, 16), jnp.float32),
            pltpu.VMEM((KB, 16), jnp.float32),
            pltpu.VMEM((KB, H), jnp.float32),
            pltpu.VMEM((KB, H), jnp.float32),
            pltpu.SemaphoreType.DMA,
            pltpu.SemaphoreType.DMA,
            pltpu.SemaphoreType.DMA,
            pltpu.SemaphoreType.DMA,
        ],
    )(_k3_agg_body)
    return k2, k3


def _ln(x, g, b):
    m = jnp.mean(x, axis=-1, keepdims=True)
    va = jnp.mean((x - m) * (x - m), axis=-1, keepdims=True)
    return (x - m) * jax.lax.rsqrt(va + 1e-5) * g + b


def _lstm(x, w, bsum):
    gates = jnp.dot(x, w, preferred_element_type=jnp.float32) + bsum
    hh = gates.shape[-1] // 4
    i = gates[:, 0:hh]
    g = gates[:, 2 * hh:3 * hh]
    o = gates[:, 3 * hh:4 * hh]
    cc = jax.nn.sigmoid(i) * jnp.tanh(g)
    return jax.nn.sigmoid(o) * jnp.tanh(cc)


def _k1(a_ref, t_ref, v_ref,
        w0f, b0f, w0r, b0r, w1f, b1f, w1r, b1r, alng, alnb,
        tw1, tb1, tlng, tlnb, tw2, tb2,
        vw1, vb1, vlng, vlnb, vw2, vb2,
        wrel, wroot, broot, x_ref, rt_ref, y_ref):
    ab = a_ref[...]
    tb = t_ref[...]
    vb = v_ref[...]
    h0 = jnp.concatenate([_lstm(ab, w0f[...], b0f[...]),
                          _lstm(ab, w0r[...], b0r[...])], axis=-1)
    h1 = jnp.concatenate([_lstm(h0, w1f[...], b1f[...]),
                          _lstm(h0, w1r[...], b1r[...])], axis=-1)
    za = _ln(h1, alng[...], alnb[...])

    th = jnp.dot(tb, tw1[...], preferred_element_type=jnp.float32) + tb1[...]
    th = jax.nn.gelu(_ln(th, tlng[...], tlnb[...]))
    zt = jnp.dot(th, tw2[...], preferred_element_type=jnp.float32) + tb2[...]

    vh = jnp.dot(vb, vw1[...], preferred_element_type=jnp.float32) + vb1[...]
    vh = jax.nn.gelu(_ln(vh, vlng[...], vlnb[...]))
    zv = jnp.dot(vh, vw2[...], preferred_element_type=jnp.float32) + vb2[...]

    x = (za + zt + zv) / 3.0
    x_ref[...] = x
    rt_ref[...] = jnp.dot(x, wroot[...], preferred_element_type=jnp.float32) + broot[...]
    for r in range(R):
        y_ref[r] = jnp.dot(x, wrel[r], preferred_element_type=jnp.float32)


def _k4(p_ref, rt_ref, xin_ref, lng, lnb, wrel, wroot, broot,
        h_ref, rt2_ref, y2_ref):
    sacc = p_ref[0] + p_ref[1] + rt_ref[...]
    h = jax.nn.gelu(_ln(sacc, lng[...], lnb[...])) + xin_ref[...]
    h_ref[...] = h
    rt2_ref[...] = jnp.dot(h, wroot[...], preferred_element_type=jnp.float32) + broot[...]
    for r in range(R):
        y2_ref[r] = jnp.dot(h, wrel[r], preferred_element_type=jnp.float32)


def _k6(p_ref, rt_ref, hin_ref, lng, lnb, bb_ref, wpool, bpool,
        out_ref, sums, maxs, cnts):
    i = pl.program_id(0)

    @pl.when(i == 0)
    def _():
        sums[...] = jnp.zeros((B, H), jnp.float32)
        maxs[...] = jnp.full((B, H), -3.0e38, jnp.float32)
        cnts[...] = jnp.zeros((B, H), jnp.float32)

    sacc = p_ref[0] + p_ref[1] + rt_ref[...]
    h2 = jax.nn.gelu(_ln(sacc, lng[...], lnb[...])) + hin_ref[...]

    bb = bb_ref[...]
    bn = bb.shape[0]
    onehot = (bb == lax.broadcasted_iota(jnp.int32, (bn, B), 1)
              ).astype(jnp.float32)
    dn = (((0,), (0,)), ((), ()))
    sums[...] += lax.dot_general(onehot, h2, dn,
                                 preferred_element_type=jnp.float32)
    cnts[...] += lax.dot_general(onehot, jnp.ones((bn, H), jnp.float32), dn,
                                 preferred_element_type=jnp.float32)

    b0 = jnp.min(bb)
    b1 = jnp.max(bb)

    def seg(b, _):
        mask = bb == b
        mval = jnp.max(jnp.where(mask, h2, -3.0e38), axis=0, keepdims=True)
        maxs[pl.ds(b, 1), :] = jnp.maximum(maxs[pl.ds(b, 1), :], mval)
        return _

    lax.fori_loop(b0, b1 + 1, seg, None)

    @pl.when(i == pl.num_programs(0) - 1)
    def _():
        cnt = cnts[...]
        mean = sums[...] / jnp.maximum(cnt, 1.0)
        mx = jnp.where(cnt > 0.0, maxs[...], 0.0)
        pooled = jnp.concatenate([mean, mx], axis=-1)
        out_ref[...] = jnp.dot(pooled, wpool[...],
                               preferred_element_type=jnp.float32) + bpool[...]


BN = 1000
_GRID = N // BN


def _full(shape):
    return pl.BlockSpec(shape, lambda i: tuple(0 for _ in shape))


def _rows(shape):
    def im(i):
        return (i,) + tuple(0 for _ in shape[1:])
    return pl.BlockSpec(shape, im)


def kernel(a, t, v, edge_index, edge_type, batch, params):
    f32 = jnp.float32
    pa, pt, pv, pg = params['audio'], params['text'], params['visual'], params['gnn']

    src = edge_index[0].astype(jnp.int32)
    dst = edge_index[1].astype(jnp.int32)
    et = edge_type.astype(jnp.int32)
    bb2 = batch.astype(jnp.int32).reshape(N, 1)

    r2 = lambda b: b.reshape(1, -1).astype(f32)

    k1_in = [
        a.astype(f32), t.astype(f32), v.astype(f32),
        pa['l0f_Wih'].T, r2(pa['l0f_bih'] + pa['l0f_bhh']),
        pa['l0r_Wih'].T, r2(pa['l0r_bih'] + pa['l0r_bhh']),
        pa['l1f_Wih'].T, r2(pa['l1f_bih'] + pa['l1f_bhh']),
        pa['l1r_Wih'].T, r2(pa['l1r_bih'] + pa['l1r_bhh']),
        r2(pa['ln_g']), r2(pa['ln_b']),
        pt['W1'], r2(pt['b1']), r2(pt['ln_g']), r2(pt['ln_b']),
        pt['W2'], r2(pt['b2']),
        pv['W1'], r2(pv['b1']), r2(pv['ln_g']), r2(pv['ln_b']),
        pv['W2'], r2(pv['b2']),
        pg['Wrel1'], pg['Wroot1'], r2(pg['b1']),
    ]
    k1_specs = [
        _rows((BN, a.shape[1])), _rows((BN, t.shape[1])), _rows((BN, v.shape[1])),
        _full((a.shape[1], 256)), _full((1, 256)),
        _full((a.shape[1], 256)), _full((1, 256)),
        _full((H, 256)), _full((1, 256)),
        _full((H, 256)), _full((1, 256)),
        _full((1, H)), _full((1, H)),
        _full((t.shape[1], 256)), _full((1, 256)), _full((1, 256)), _full((1, 256)),
        _full((256, H)), _full((1, H)),
        _full((v.shape[1], H)), _full((1, H)), _full((1, H)), _full((1, H)),
        _full((H, H)), _full((1, H)),
        _full((R, H, H)), _full((H, H)), _full((1, H)),
    ]
    x, rt1, y1 = pl.pallas_call(
        _k1,
        grid=(_GRID,),
        in_specs=k1_specs,
        out_specs=[_rows((BN, H)), _rows((BN, H)),
                   pl.BlockSpec((R, BN, H), lambda i: (0, i, 0))],
        out_shape=[jax.ShapeDtypeStruct((N, H), f32),
                   jax.ShapeDtypeStruct((N, H), f32),
                   jax.ShapeDtypeStruct((R, N, H), f32)],
    )(*k1_in)

    _k2_norm, _k3_agg = _sc_kernels()
    zseg = jnp.zeros((HBINS // NS, 16), f32)
    ones16 = jnp.ones((128, 16), f32)
    norm, gi = _k2_norm(dst, et, src, zseg, ones16)

    zslab = jnp.zeros((SLAB, H), f32)

    gi2 = gi.reshape(E // KB, KB)
    dst2 = dst.reshape(E // KB, KB)
    p1 = _k3_agg(y1.reshape(R * N, H), gi2, dst2, norm, zslab)

    k4_specs = [
        pl.BlockSpec((NC, BN, H), lambda i: (0, i, 0)),
        _rows((BN, H)), _rows((BN, H)),
        _full((1, H)), _full((1, H)),
        _full((R, H, H)), _full((H, H)), _full((1, H)),
    ]
    h, rt2, y2 = pl.pallas_call(
        _k4,
        grid=(_GRID,),
        in_specs=k4_specs,
        out_specs=[_rows((BN, H)), _rows((BN, H)),
                   pl.BlockSpec((R, BN, H), lambda i: (0, i, 0))],
        out_shape=[jax.ShapeDtypeStruct((N, H), f32),
                   jax.ShapeDtypeStruct((N, H), f32),
                   jax.ShapeDtypeStruct((R, N, H), f32)],
    )(p1.reshape(NC, N, H), rt1, x, r2(pg['ln1_g']), r2(pg['ln1_b']),
      pg['Wrel2'], pg['Wroot2'], r2(pg['b2']))

    p2 = _k3_agg(y2.reshape(R * N, H), gi2, dst2, norm, zslab)

    k6_specs = [
        pl.BlockSpec((NC, BN, H), lambda i: (0, i, 0)),
        _rows((BN, H)), _rows((BN, H)),
        _full((1, H)), _full((1, H)),
        _rows((BN, 1)),
        _full((2 * H, H)), _full((1, H)),
    ]
    out = pl.pallas_call(
        _k6,
        grid=(_GRID,),
        in_specs=k6_specs,
        out_specs=pl.BlockSpec((B, H), lambda i: (0, 0)),
        out_shape=jax.ShapeDtypeStruct((B, H), f32),
        scratch_shapes=[pltpu.VMEM((B, H), f32)] * 3,
    )(p2.reshape(NC, N, H), rt2, h, r2(pg['ln2_g']), r2(pg['ln2_b']),
      bb2, pg['Wpool'], r2(pg['bpool']))

    return out
